# baseline (speedup 1.0000x reference)
import functools

import jax
import jax.numpy as jnp
from jax import lax
from jax.experimental import pallas as pl
from jax.experimental.pallas import tpu as pltpu
from jax.experimental.pallas import tpu_sc as plsc

H = 768
EPS = 1e-12
C = 32
NBUF = 4


def _make_sc_gather(B, S, V):
    info = plsc.get_sparse_core_info()
    NC, NS = info.num_cores, info.num_subcores
    NW = NC * NS
    TPW = (B * S) // NW
    WPR = S // TPW
    NCHUNK = TPW // C
    LAG = NBUF - 1

    mesh = plsc.VectorSubcoreMesh(core_axis_name="c", subcore_axis_name="s")

    @functools.partial(
        pl.kernel,
        mesh=mesh,
        out_type=jax.ShapeDtypeStruct((B, S, H), jnp.float32),
        scratch_types=[
            pltpu.VMEM((TPW,), jnp.int32),
        ] + [pltpu.VMEM((C, H), jnp.float32) for _ in range(NBUF)]
          + [pltpu.SemaphoreType.DMA for _ in range(2 * NBUF)],
    )
    def k(ids_hbm, word_hbm, out_hbm, idx_all, *rest):
        bufs = rest[:NBUF]
        gsem = rest[NBUF:2 * NBUF]
        ssem = rest[2 * NBUF:]
        wid = lax.axis_index("s") * NC + lax.axis_index("c")
        row = wid // WPR
        col0 = (wid % WPR) * TPW
        pltpu.sync_copy(ids_hbm.at[row, pl.ds(col0, TPW)], idx_all)

        gathers = [None] * NCHUNK
        scatters = [None] * NCHUNK

        def start_scatter(d):
            gathers[d].wait()
            scatters[d] = pltpu.async_copy(
                bufs[d % NBUF],
                out_hbm.at[row, pl.ds(col0 + d * C, C), :],
                ssem[d % NBUF])

        for c in range(NCHUNK):
            r = c % NBUF
            if c >= NBUF:
                scatters[c - NBUF].wait()
            gathers[c] = pltpu.async_copy(
                word_hbm.at[idx_all.at[pl.ds(c * C, C)]], bufs[r], gsem[r])
            if c >= LAG:
                start_scatter(c - LAG)
        for d in range(NCHUNK - LAG, NCHUNK):
            start_scatter(d)
        for d in range(NCHUNK - NBUF, NCHUNK):
            scatters[d].wait()

    return k


def _make_tc_ln(BT, S, B, T):
    TOK = 2048
    SCH = S // TOK
    grid = (SCH, B)

    def body(g_ref, pos_ref, tt_ref, type_ref, gam_ref, bet_ref, o_ref):
        x = g_ref[0] + pos_ref[...]
        tt = tt_ref[...]
        t0 = type_ref[0, :][None, :]
        t1 = type_ref[1, :][None, :]
        x = x + jnp.where(tt == 0, t0, t1)
        mean = jnp.mean(x, axis=-1, keepdims=True)
        c = x - mean
        var = jnp.mean(c * c, axis=-1, keepdims=True)
        inv = lax.rsqrt(var + EPS)
        o_ref[0] = gam_ref[...] * (c * inv) + bet_ref[...]

    return pl.pallas_call(
        body,
        grid=grid,
        in_specs=[
            pl.BlockSpec((1, TOK, H), lambda sc, b: (b, sc, 0)),
            pl.BlockSpec((TOK, H), lambda sc, b: (sc, 0)),
            pl.BlockSpec((TOK, 1), lambda sc, b: (b * SCH + sc, 0)),
            pl.BlockSpec((T, H), lambda sc, b: (0, 0)),
            pl.BlockSpec((1, H), lambda sc, b: (0, 0)),
            pl.BlockSpec((1, H), lambda sc, b: (0, 0)),
        ],
        out_specs=pl.BlockSpec((1, TOK, H), lambda sc, b: (b, sc, 0)),
        out_shape=jax.ShapeDtypeStruct((B, S, H), jnp.float32),
    )


def kernel(input_ids, token_type_ids, word_emb, pos_emb, type_emb, gamma, beta):
    B, S = input_ids.shape
    V = word_emb.shape[0]
    T = type_emb.shape[0]
    BT = B * S
    NSLICE = 2
    BSL = B // NSLICE
    ids = input_ids.astype(jnp.int32)
    tts = token_type_ids.astype(jnp.int32)
    sc = _make_sc_gather(BSL, S, V)
    tc = _make_tc_ln(BSL * S, S, BSL, T)
    g1 = gamma.reshape(1, H)
    b1 = beta.reshape(1, H)
    gathered = [sc(lax.slice_in_dim(ids, s * BSL, (s + 1) * BSL), word_emb)
                for s in range(NSLICE)]
    outs = [tc(gathered[s], pos_emb,
               lax.slice_in_dim(tts, s * BSL, (s + 1) * BSL).reshape(BSL * S, 1),
               type_emb, g1, b1)
            for s in range(NSLICE)]
    return jnp.concatenate(outs, axis=0)


# speedup vs baseline: 1.2870x; 1.2870x over previous
import functools

import jax
import jax.numpy as jnp
from jax import lax
from jax.experimental import pallas as pl
from jax.experimental.pallas import tpu as pltpu
from jax.experimental.pallas import tpu_sc as plsc

H = 768
EPS = 1e-12
C = 64
NBUF = 2


def _make_sc_gather(B, S, V):
    info = plsc.get_sparse_core_info()
    NC, NS = info.num_cores, info.num_subcores
    NW = NC * NS
    TPW = (B * S) // NW
    WPR = S // TPW
    NCHUNK = TPW // C
    LAG = NBUF - 1

    mesh = plsc.VectorSubcoreMesh(core_axis_name="c", subcore_axis_name="s")

    @functools.partial(
        pl.kernel,
        mesh=mesh,
        out_type=jax.ShapeDtypeStruct((B, S, H), jnp.float32),
        scratch_types=[
            pltpu.VMEM((TPW,), jnp.int32),
        ] + [pltpu.VMEM((C, H), jnp.float32) for _ in range(NBUF)]
          + [pltpu.SemaphoreType.DMA for _ in range(2 * NBUF)],
    )
    def k(ids_hbm, word_hbm, out_hbm, idx_all, *rest):
        bufs = rest[:NBUF]
        gsem = rest[NBUF:2 * NBUF]
        ssem = rest[2 * NBUF:]
        wid = lax.axis_index("s") * NC + lax.axis_index("c")
        row = wid // WPR
        col0 = (wid % WPR) * TPW
        pltpu.sync_copy(ids_hbm.at[row, pl.ds(col0, TPW)], idx_all)

        gathers = [None] * NCHUNK
        scatters = [None] * NCHUNK

        def start_scatter(d):
            gathers[d].wait()
            scatters[d] = pltpu.async_copy(
                bufs[d % NBUF],
                out_hbm.at[row, pl.ds(col0 + d * C, C), :],
                ssem[d % NBUF])

        for c in range(NCHUNK):
            r = c % NBUF
            if c >= NBUF:
                scatters[c - NBUF].wait()
            gathers[c] = pltpu.async_copy(
                word_hbm.at[idx_all.at[pl.ds(c * C, C)]], bufs[r], gsem[r])
            if c >= LAG:
                start_scatter(c - LAG)
        for d in range(NCHUNK - LAG, NCHUNK):
            start_scatter(d)
        for d in range(NCHUNK - NBUF, NCHUNK):
            scatters[d].wait()

    return k


def _make_tc_ln(BT, S, B, T):
    TOK = 2048
    SCH = S // TOK
    grid = (SCH, B)

    def body(g_ref, pos_ref, tt_ref, type_ref, gam_ref, bet_ref, o_ref):
        x = g_ref[0] + pos_ref[...]
        tt = tt_ref[...]
        t0 = type_ref[0, :][None, :]
        t1 = type_ref[1, :][None, :]
        x = x + jnp.where(tt == 0, t0, t1)
        mean = jnp.mean(x, axis=-1, keepdims=True)
        c = x - mean
        var = jnp.mean(c * c, axis=-1, keepdims=True)
        inv = lax.rsqrt(var + EPS)
        o_ref[0] = gam_ref[...] * (c * inv) + bet_ref[...]

    return pl.pallas_call(
        body,
        grid=grid,
        in_specs=[
            pl.BlockSpec((1, TOK, H), lambda sc, b: (b, sc, 0)),
            pl.BlockSpec((TOK, H), lambda sc, b: (sc, 0)),
            pl.BlockSpec((TOK, 1), lambda sc, b: (b * SCH + sc, 0)),
            pl.BlockSpec((T, H), lambda sc, b: (0, 0)),
            pl.BlockSpec((1, H), lambda sc, b: (0, 0)),
            pl.BlockSpec((1, H), lambda sc, b: (0, 0)),
        ],
        out_specs=pl.BlockSpec((1, TOK, H), lambda sc, b: (b, sc, 0)),
        out_shape=jax.ShapeDtypeStruct((B, S, H), jnp.float32),
    )


def kernel(input_ids, token_type_ids, word_emb, pos_emb, type_emb, gamma, beta):
    B, S = input_ids.shape
    V = word_emb.shape[0]
    T = type_emb.shape[0]
    BT = B * S
    gathered = _make_sc_gather(B, S, V)(input_ids.astype(jnp.int32), word_emb)
    out = _make_tc_ln(BT, S, B, T)(
        gathered, pos_emb,
        token_type_ids.astype(jnp.int32).reshape(BT, 1), type_emb,
        gamma.reshape(1, H), beta.reshape(1, H))
    return out


# speedup vs baseline: 1.3227x; 1.0278x over previous
import functools

import jax
import jax.numpy as jnp
from jax import lax
from jax.experimental import pallas as pl
from jax.experimental.pallas import tpu as pltpu
from jax.experimental.pallas import tpu_sc as plsc

H = 768
EPS = 1e-12
C = 32
NBUF = 5


def _make_sc_gather(B, S, V):
    info = plsc.get_sparse_core_info()
    NC, NS = info.num_cores, info.num_subcores
    NW = NC * NS
    TPW = (B * S) // NW
    WPR = S // TPW
    NCHUNK = TPW // C
    LAG = NBUF - 1

    mesh = plsc.VectorSubcoreMesh(core_axis_name="c", subcore_axis_name="s")

    @functools.partial(
        pl.kernel,
        mesh=mesh,
        out_type=jax.ShapeDtypeStruct((B, S, H), jnp.float32),
        scratch_types=[
            pltpu.VMEM((TPW,), jnp.int32),
        ] + [pltpu.VMEM((C, H), jnp.float32) for _ in range(NBUF)]
          + [pltpu.SemaphoreType.DMA for _ in range(2 * NBUF)],
    )
    def k(ids_hbm, word_hbm, out_hbm, idx_all, *rest):
        bufs = rest[:NBUF]
        gsem = rest[NBUF:2 * NBUF]
        ssem = rest[2 * NBUF:]
        wid = lax.axis_index("s") * NC + lax.axis_index("c")
        row = wid // WPR
        col0 = (wid % WPR) * TPW
        pltpu.sync_copy(ids_hbm.at[row, pl.ds(col0, TPW)], idx_all)

        gathers = [None] * NCHUNK
        scatters = [None] * NCHUNK

        def start_scatter(d):
            gathers[d].wait()
            scatters[d] = pltpu.async_copy(
                bufs[d % NBUF],
                out_hbm.at[row, pl.ds(col0 + d * C, C), :],
                ssem[d % NBUF])

        for c in range(NCHUNK):
            r = c % NBUF
            if c >= NBUF:
                scatters[c - NBUF].wait()
            gathers[c] = pltpu.async_copy(
                word_hbm.at[idx_all.at[pl.ds(c * C, C)]], bufs[r], gsem[r])
            if c >= LAG:
                start_scatter(c - LAG)
        for d in range(NCHUNK - LAG, NCHUNK):
            start_scatter(d)
        for d in range(NCHUNK - NBUF, NCHUNK):
            scatters[d].wait()

    return k


def _make_tc_ln(BT, S, B, T):
    TOK = 2048
    SCH = S // TOK
    grid = (SCH, B)

    def body(g_ref, pos_ref, tt_ref, type_ref, gam_ref, bet_ref, o_ref):
        x = g_ref[0] + pos_ref[...]
        tt = tt_ref[...]
        t0 = type_ref[0, :][None, :]
        t1 = type_ref[1, :][None, :]
        x = x + jnp.where(tt == 0, t0, t1)
        mean = jnp.mean(x, axis=-1, keepdims=True)
        c = x - mean
        var = jnp.mean(c * c, axis=-1, keepdims=True)
        inv = lax.rsqrt(var + EPS)
        o_ref[0] = gam_ref[...] * (c * inv) + bet_ref[...]

    return pl.pallas_call(
        body,
        grid=grid,
        in_specs=[
            pl.BlockSpec((1, TOK, H), lambda sc, b: (b, sc, 0)),
            pl.BlockSpec((TOK, H), lambda sc, b: (sc, 0)),
            pl.BlockSpec((TOK, 1), lambda sc, b: (b * SCH + sc, 0)),
            pl.BlockSpec((T, H), lambda sc, b: (0, 0)),
            pl.BlockSpec((1, H), lambda sc, b: (0, 0)),
            pl.BlockSpec((1, H), lambda sc, b: (0, 0)),
        ],
        out_specs=pl.BlockSpec((1, TOK, H), lambda sc, b: (b, sc, 0)),
        out_shape=jax.ShapeDtypeStruct((B, S, H), jnp.float32),
    )


def kernel(input_ids, token_type_ids, word_emb, pos_emb, type_emb, gamma, beta):
    B, S = input_ids.shape
    V = word_emb.shape[0]
    T = type_emb.shape[0]
    BT = B * S
    gathered = _make_sc_gather(B, S, V)(input_ids.astype(jnp.int32), word_emb)
    out = _make_tc_ln(BT, S, B, T)(
        gathered, pos_emb,
        token_type_ids.astype(jnp.int32).reshape(BT, 1), type_emb,
        gamma.reshape(1, H), beta.reshape(1, H))
    return out
